# Initial kernel scaffold; baseline (speedup 1.0000x reference)
import jax
import jax.numpy as jnp
from jax.experimental import pallas as pl


def kernel(mailbox_h, W, b):
    raise NotImplementedError("write your pallas kernel here")


# speedup vs baseline: 1.1474x; 1.1474x over previous
import functools

import jax
import jax.numpy as jnp
from jax import lax
from jax.experimental import pallas as pl

N = 10000
DEG = 32
IN_FEATS = 128
OUT_FEATS = 128

BN = 400


def _body(x_ref, w_ref, b_ref, o_ref):
    x = x_ref[...]
    h = jnp.sum(x, axis=1) * (1.0 / DEG)
    o = lax.dot_general(h, w_ref[...], (((1,), (1,)), ((), ())),
                        preferred_element_type=jnp.float32)
    o_ref[...] = o + b_ref[...]


@functools.partial(jax.jit, static_argnames=())
def kernel(mailbox_h, W, b):
    b2 = b.reshape(1, OUT_FEATS)
    grid = (N // BN,)
    out = pl.pallas_call(
        _body,
        grid=grid,
        in_specs=[
            pl.BlockSpec((BN, DEG, IN_FEATS), lambda i: (i, 0, 0)),
            pl.BlockSpec((OUT_FEATS, IN_FEATS), lambda i: (0, 0)),
            pl.BlockSpec((1, OUT_FEATS), lambda i: (0, 0)),
        ],
        out_specs=pl.BlockSpec((BN, OUT_FEATS), lambda i: (i, 0)),
        out_shape=jax.ShapeDtypeStruct((N, OUT_FEATS), jnp.float32),
    )(mailbox_h, W, b2)
    return out
